# Initial kernel scaffold; baseline (speedup 1.0000x reference)
import functools

import jax
import jax.numpy as jnp
from jax import lax
from jax.experimental import pallas as pl
from jax.experimental.pallas import tpu as pltpu
from jax.experimental.pallas import tpu_sc as plsc

N_NODES = 10000
N_EDGES = 320000
IN_DIM = 128
HID = 16
HEADS = 8
OUT_DIM = 64

CHUNK = 128
NTILES = 32
NPAD = 10240
ROWS_PER_TILE = NPAD // 16
NTAB = N_NODES + 16

E_TOT = N_EDGES + N_NODES
E_PAD = ((E_TOT + CHUNK * NTILES - 1) // (CHUNK * NTILES)) * (CHUNK * NTILES)
EDGES_PER_TILE = E_PAD // NTILES
NCHUNKS = EDGES_PER_TILE // CHUNK


def _sc_edge_layer(C):
  nvec = C // 16
  mesh = plsc.VectorSubcoreMesh(core_axis_name="c", subcore_axis_name="s")

  @functools.partial(
      pl.kernel,
      out_type=[
          jax.ShapeDtypeStruct((2, NPAD, C), jnp.float32),
          jax.ShapeDtypeStruct((2, NPAD, 16), jnp.float32),
      ],
      mesh=mesh,
      scratch_types=[
          pltpu.VMEM((CHUNK,), jnp.int32),
          pltpu.VMEM((CHUNK,), jnp.int32),
          pltpu.VMEM((CHUNK, 16), jnp.float32),
          pltpu.VMEM((CHUNK, 16), jnp.float32),
          pltpu.VMEM((CHUNK, 16), jnp.float32),
          pltpu.VMEM((CHUNK, C), jnp.float32),
          pltpu.VMEM((2, 16), jnp.float32),
          pltpu.VMEM_SHARED((NPAD, C), jnp.float32),
          pltpu.VMEM_SHARED((NPAD, 16), jnp.float32),
          pltpu.SemaphoreType.DMA,
          pltpu.SemaphoreType.DMA,
          pltpu.SemaphoreType.DMA,
      ],
  )
  def k(h_hbm, asrc_hbm, adst_hbm, m_hbm, src_hbm, dst_hbm, zc_hbm, z16_hbm,
        acc_out, den_out,
        idxs, idxd, asr, adr, wbuf, hrows, mbuf, acc_sh, den_sh,
        sem_a, sem_b, sem_h):
    cid = lax.axis_index("c")
    sid = lax.axis_index("s")
    row0 = sid * ROWS_PER_TILE

    pltpu.sync_copy(zc_hbm.at[pl.ds(row0, ROWS_PER_TILE)],
                    acc_sh.at[pl.ds(row0, ROWS_PER_TILE)])
    pltpu.sync_copy(z16_hbm.at[pl.ds(row0, ROWS_PER_TILE)],
                    den_sh.at[pl.ds(row0, ROWS_PER_TILE)])
    pltpu.sync_copy(m_hbm, mbuf)
    mvec = mbuf[0, :] + mbuf[1, :]
    plsc.subcore_barrier()

    wid = sid * 2 + cid
    base = wid * EDGES_PER_TILE

    def chunk_body(i, carry):
      cb = base + i * CHUNK
      pltpu.sync_copy(src_hbm.at[pl.ds(cb, CHUNK)], idxs)
      pltpu.sync_copy(dst_hbm.at[pl.ds(cb, CHUNK)], idxd)
      cp_a = pltpu.async_copy(asrc_hbm.at[idxs], asr, sem_a)
      cp_b = pltpu.async_copy(adst_hbm.at[idxd], adr, sem_b)
      cp_h = pltpu.async_copy(h_hbm.at[idxs], hrows, sem_h)
      cp_a.wait()
      cp_b.wait()

      def wfun(e, c2):
        v = asr[e, :] + adr[e, :]
        v = jnp.where(v > 0, v, 0.2 * v)
        wbuf[e, :] = jnp.exp(v - mvec)
        return c2

      lax.fori_loop(0, CHUNK, wfun, 0, unroll=4)
      pltpu.sync_copy(wbuf, den_sh.at[idxd], add=True)
      cp_h.wait()

      def sfun(e, c2):
        for j in range(nvec):
          w = wbuf[e, j] if C == 128 else wbuf[e, 0]
          hrows[e, pl.ds(j * 16, 16)] = hrows[e, pl.ds(j * 16, 16)] * w
        return c2

      lax.fori_loop(0, CHUNK, sfun, 0, unroll=2)
      pltpu.sync_copy(hrows, acc_sh.at[idxd], add=True)
      return carry

    lax.fori_loop(0, NCHUNKS, chunk_body, 0)
    plsc.subcore_barrier()
    pltpu.sync_copy(acc_sh.at[pl.ds(row0, ROWS_PER_TILE)],
                    acc_out.at[cid, pl.ds(row0, ROWS_PER_TILE)])
    pltpu.sync_copy(den_sh.at[pl.ds(row0, ROWS_PER_TILE)],
                    den_out.at[cid, pl.ds(row0, ROWS_PER_TILE)])

  return k


_edge128 = _sc_edge_layer(128)
_edge64 = _sc_edge_layer(64)

BLK = 1000
GRID = N_NODES // BLK


def _tc_a_body(x_ref, w1_ref, a1s_ref, a1d_ref,
               h_ref, asrc_ref, adst_ref, m_ref):
  h = jnp.dot(x_ref[...], w1_ref[...], preferred_element_type=jnp.float32)
  h_ref[...] = h
  asrc = jnp.dot(h, a1s_ref[...], preferred_element_type=jnp.float32)
  adst = jnp.dot(h, a1d_ref[...], preferred_element_type=jnp.float32)
  asrc_ref[...] = asrc
  adst_ref[...] = adst
  cur = jnp.concatenate([jnp.max(asrc, axis=0, keepdims=True),
                         jnp.max(adst, axis=0, keepdims=True)], axis=0)

  @pl.when(pl.program_id(0) == 0)
  def _():
    m_ref[...] = cur

  @pl.when(pl.program_id(0) != 0)
  def _():
    m_ref[...] = jnp.maximum(m_ref[...], cur)


def _tc_b_body(accA_ref, accB_ref, denA_ref, denB_ref, e1_ref, b1_ref,
               w2_ref, a2s_ref, a2d_ref,
               h2_ref, asrc_ref, adst_ref, m_ref):
  den = denA_ref[...] + denB_ref[...] + 1e-16
  dexp = jnp.dot(den, e1_ref[...], preferred_element_type=jnp.float32)
  out1 = (accA_ref[...] + accB_ref[...]) / dexp + b1_ref[...]
  out1 = jnp.where(out1 > 0, out1, jnp.expm1(jnp.minimum(out1, 0.0)))
  h2 = jnp.dot(out1, w2_ref[...], preferred_element_type=jnp.float32)
  h2_ref[...] = h2
  asrc = jnp.dot(h2, a2s_ref[...], preferred_element_type=jnp.float32)
  adst = jnp.dot(h2, a2d_ref[...], preferred_element_type=jnp.float32)
  asrc_ref[...] = asrc
  adst_ref[...] = adst
  cur = jnp.concatenate([jnp.max(asrc, axis=0, keepdims=True),
                         jnp.max(adst, axis=0, keepdims=True)], axis=0)

  @pl.when(pl.program_id(0) == 0)
  def _():
    m_ref[...] = cur

  @pl.when(pl.program_id(0) != 0)
  def _():
    m_ref[...] = jnp.maximum(m_ref[...], cur)


def _tc_c_body(accA_ref, accB_ref, denA_ref, denB_ref, e2_ref, b2_ref,
               out_ref):
  den = denA_ref[...] + denB_ref[...] + 1e-16
  dexp = jnp.dot(den, e2_ref[...], preferred_element_type=jnp.float32)
  out_ref[...] = (accA_ref[...] + accB_ref[...]) / dexp + b2_ref[...]


def _full_spec(shape):
  return pl.BlockSpec(shape, lambda i: (0,) * len(shape))


def _row_spec(cols):
  return pl.BlockSpec((BLK, cols), lambda i: (i, 0))


def _blockdiag(att):
  H, Cc = att.shape
  eye = jnp.eye(16, dtype=att.dtype)[:H]
  return (att[:, :, None] * eye[:, None, :]).reshape(H * Cc, 16)


def kernel(x, edge_index, W1, att_src1, att_dst1, b1,
           W2, att_src2, att_dst2, b2):
  f32 = jnp.float32
  ar = jnp.arange(N_NODES, dtype=jnp.int32)
  padv = jnp.full((E_PAD - E_TOT,), N_NODES, dtype=jnp.int32)
  src = jnp.concatenate([edge_index[0].astype(jnp.int32), ar, padv])
  dst = jnp.concatenate([edge_index[1].astype(jnp.int32), ar, padv])

  A1s = _blockdiag(att_src1)
  A1d = _blockdiag(att_dst1)
  A2s = _blockdiag(att_src2)
  A2d = _blockdiag(att_dst2)
  E1 = jnp.concatenate([jnp.kron(jnp.eye(8, dtype=f32), jnp.ones((1, 16), f32)),
                        jnp.zeros((8, 128), f32)], axis=0)
  E2 = jnp.concatenate([jnp.ones((1, 64), f32),
                        jnp.zeros((15, 64), f32)], axis=0)
  z128 = jnp.zeros((NPAD, 128), f32)
  z64 = jnp.zeros((NPAD, 64), f32)
  z16 = jnp.zeros((NPAD, 16), f32)

  h1, asrc1, adst1, m1 = pl.pallas_call(
      _tc_a_body,
      grid=(GRID,),
      in_specs=[_row_spec(128), _full_spec((128, 128)),
                _full_spec((128, 16)), _full_spec((128, 16))],
      out_specs=[_row_spec(128), _row_spec(16), _row_spec(16),
                 _full_spec((2, 16))],
      out_shape=[jax.ShapeDtypeStruct((N_NODES, 128), f32),
                 jax.ShapeDtypeStruct((N_NODES, 16), f32),
                 jax.ShapeDtypeStruct((N_NODES, 16), f32),
                 jax.ShapeDtypeStruct((2, 16), f32)],
  )(x, W1, A1s, A1d)

  pad16 = ((0, 16), (0, 0))
  acc1, den1 = _edge128(jnp.pad(h1, pad16), jnp.pad(asrc1, pad16),
                        jnp.pad(adst1, pad16), m1, src, dst, z128, z16)

  h2, asrc2, adst2, m2 = pl.pallas_call(
      _tc_b_body,
      grid=(GRID,),
      in_specs=[_row_spec(128), _row_spec(128), _row_spec(16), _row_spec(16),
                _full_spec((16, 128)), _full_spec((1, 128)),
                _full_spec((128, 64)), _full_spec((64, 16)),
                _full_spec((64, 16))],
      out_specs=[_row_spec(64), _row_spec(16), _row_spec(16),
                 _full_spec((2, 16))],
      out_shape=[jax.ShapeDtypeStruct((N_NODES, 64), f32),
                 jax.ShapeDtypeStruct((N_NODES, 16), f32),
                 jax.ShapeDtypeStruct((N_NODES, 16), f32),
                 jax.ShapeDtypeStruct((2, 16), f32)],
  )(acc1[0, :N_NODES], acc1[1, :N_NODES],
    den1[0, :N_NODES], den1[1, :N_NODES],
    E1, b1.reshape(1, 128), W2, A2s, A2d)

  acc2, den2 = _edge64(jnp.pad(h2, pad16), jnp.pad(asrc2, pad16),
                       jnp.pad(adst2, pad16), m2, src, dst, z64, z16)

  out = pl.pallas_call(
      _tc_c_body,
      grid=(GRID,),
      in_specs=[_row_spec(64), _row_spec(64), _row_spec(16), _row_spec(16),
                _full_spec((16, 64)), _full_spec((1, 64))],
      out_specs=_row_spec(64),
      out_shape=jax.ShapeDtypeStruct((N_NODES, 64), f32),
  )(acc2[0, :N_NODES], acc2[1, :N_NODES],
    den2[0, :N_NODES], den2[1, :N_NODES],
    E2, b2.reshape(1, 64))
  return out


# speedup vs baseline: 47.3865x; 47.3865x over previous
import functools

import jax
import jax.numpy as jnp
from jax import lax
from jax.experimental import pallas as pl
from jax.experimental.pallas import tpu as pltpu
from jax.experimental.pallas import tpu_sc as plsc

N_NODES = 10000
N_EDGES = 320000
IN_DIM = 128
HID = 16
HEADS = 8
OUT_DIM = 64

CHUNK = 128
NTILES = 32
NPAD = 10240
ROWS_PER_TILE = NPAD // 16
NTAB = N_NODES + 16

E_TOT = N_EDGES + N_NODES
E_PAD = ((E_TOT + CHUNK * NTILES - 1) // (CHUNK * NTILES)) * (CHUNK * NTILES)
EDGES_PER_TILE = E_PAD // NTILES
NCHUNKS = EDGES_PER_TILE // CHUNK


def _sc_edge_layer(C):
  nvec = C // 16
  mesh = plsc.VectorSubcoreMesh(core_axis_name="c", subcore_axis_name="s")

  @functools.partial(
      pl.kernel,
      out_type=[
          jax.ShapeDtypeStruct((2, NPAD, C), jnp.float32),
          jax.ShapeDtypeStruct((2, NPAD, 16), jnp.float32),
      ],
      mesh=mesh,
      compiler_params=pltpu.CompilerParams(use_tc_tiling_on_sc=False),
      scratch_types=[
          pltpu.VMEM((CHUNK,), jnp.int32),
          pltpu.VMEM((CHUNK,), jnp.int32),
          pltpu.VMEM((CHUNK, 16), jnp.float32),
          pltpu.VMEM((CHUNK, 16), jnp.float32),
          pltpu.VMEM((CHUNK, 16), jnp.float32),
          pltpu.VMEM((CHUNK, C), jnp.float32),
          pltpu.VMEM((2, 16), jnp.float32),
          pltpu.VMEM_SHARED((NPAD, C), jnp.float32),
          pltpu.VMEM_SHARED((NPAD, 16), jnp.float32),
          pltpu.SemaphoreType.DMA,
          pltpu.SemaphoreType.DMA,
          pltpu.SemaphoreType.DMA,
      ],
  )
  def k(h_hbm, asrc_hbm, adst_hbm, m_hbm, src_hbm, dst_hbm, zc_hbm, z16_hbm,
        acc_out, den_out,
        idxs, idxd, asr, adr, wbuf, hrows, mbuf, acc_sh, den_sh,
        sem_a, sem_b, sem_h):
    cid = lax.axis_index("c")
    sid = lax.axis_index("s")
    row0 = sid * ROWS_PER_TILE

    pltpu.sync_copy(zc_hbm.at[pl.ds(row0, ROWS_PER_TILE)],
                    acc_sh.at[pl.ds(row0, ROWS_PER_TILE)])
    pltpu.sync_copy(z16_hbm.at[pl.ds(row0, ROWS_PER_TILE)],
                    den_sh.at[pl.ds(row0, ROWS_PER_TILE)])
    pltpu.sync_copy(m_hbm, mbuf)
    mvec = mbuf[0, :] + mbuf[1, :]
    plsc.subcore_barrier()

    wid = sid * 2 + cid
    base = wid * EDGES_PER_TILE

    def chunk_body(i, carry):
      cb = base + i * CHUNK
      pltpu.sync_copy(src_hbm.at[pl.ds(cb, CHUNK)], idxs)
      pltpu.sync_copy(dst_hbm.at[pl.ds(cb, CHUNK)], idxd)
      cp_a = pltpu.async_copy(asrc_hbm.at[idxs], asr, sem_a)
      cp_b = pltpu.async_copy(adst_hbm.at[idxd], adr, sem_b)
      cp_h = pltpu.async_copy(h_hbm.at[idxs], hrows, sem_h)
      cp_a.wait()
      cp_b.wait()

      def wfun(e, c2):
        v = asr[e, :] + adr[e, :]
        v = jnp.where(v > 0, v, 0.2 * v)
        wbuf[e, :] = jnp.exp(v - mvec)
        return c2

      lax.fori_loop(0, CHUNK, wfun, 0, unroll=4)
      pltpu.sync_copy(wbuf, den_sh.at[idxd], add=True)
      cp_h.wait()

      def sfun(e, c2):
        wv = wbuf[e, :]
        for j in range(nvec):
          w = wv[j] if C == 128 else wv[0]
          hrows[e, pl.ds(j * 16, 16)] = hrows[e, pl.ds(j * 16, 16)] * w
        return c2

      lax.fori_loop(0, CHUNK, sfun, 0, unroll=2)
      pltpu.sync_copy(hrows, acc_sh.at[idxd], add=True)
      return carry

    lax.fori_loop(0, NCHUNKS, chunk_body, 0)
    plsc.subcore_barrier()
    pltpu.sync_copy(acc_sh.at[pl.ds(row0, ROWS_PER_TILE)],
                    acc_out.at[cid, pl.ds(row0, ROWS_PER_TILE)])
    pltpu.sync_copy(den_sh.at[pl.ds(row0, ROWS_PER_TILE)],
                    den_out.at[cid, pl.ds(row0, ROWS_PER_TILE)])

  return k


_edge128 = _sc_edge_layer(128)
_edge64 = _sc_edge_layer(64)

BLK = 1000
GRID = N_NODES // BLK


def _tc_a_body(x_ref, w1_ref, a1s_ref, a1d_ref,
               h_ref, asrc_ref, adst_ref, m_ref):
  h = jnp.dot(x_ref[...], w1_ref[...], preferred_element_type=jnp.float32)
  h_ref[...] = h
  asrc = jnp.dot(h, a1s_ref[...], preferred_element_type=jnp.float32)
  adst = jnp.dot(h, a1d_ref[...], preferred_element_type=jnp.float32)
  asrc_ref[...] = asrc
  adst_ref[...] = adst
  cur = jnp.concatenate([jnp.max(asrc, axis=0, keepdims=True),
                         jnp.max(adst, axis=0, keepdims=True)], axis=0)

  @pl.when(pl.program_id(0) == 0)
  def _():
    m_ref[...] = cur

  @pl.when(pl.program_id(0) != 0)
  def _():
    m_ref[...] = jnp.maximum(m_ref[...], cur)


def _tc_b_body(accA_ref, accB_ref, denA_ref, denB_ref, e1_ref, b1_ref,
               w2_ref, a2s_ref, a2d_ref,
               h2_ref, asrc_ref, adst_ref, m_ref):
  den = denA_ref[...] + denB_ref[...] + 1e-16
  dexp = jnp.dot(den, e1_ref[...], preferred_element_type=jnp.float32)
  out1 = (accA_ref[...] + accB_ref[...]) / dexp + b1_ref[...]
  out1 = jnp.where(out1 > 0, out1, jnp.exp(jnp.minimum(out1, 0.0)) - 1.0)
  h2 = jnp.dot(out1, w2_ref[...], preferred_element_type=jnp.float32)
  h2_ref[...] = h2
  asrc = jnp.dot(h2, a2s_ref[...], preferred_element_type=jnp.float32)
  adst = jnp.dot(h2, a2d_ref[...], preferred_element_type=jnp.float32)
  asrc_ref[...] = asrc
  adst_ref[...] = adst
  cur = jnp.concatenate([jnp.max(asrc, axis=0, keepdims=True),
                         jnp.max(adst, axis=0, keepdims=True)], axis=0)

  @pl.when(pl.program_id(0) == 0)
  def _():
    m_ref[...] = cur

  @pl.when(pl.program_id(0) != 0)
  def _():
    m_ref[...] = jnp.maximum(m_ref[...], cur)


def _tc_c_body(accA_ref, accB_ref, denA_ref, denB_ref, e2_ref, b2_ref,
               out_ref):
  den = denA_ref[...] + denB_ref[...] + 1e-16
  dexp = jnp.dot(den, e2_ref[...], preferred_element_type=jnp.float32)
  out_ref[...] = (accA_ref[...] + accB_ref[...]) / dexp + b2_ref[...]


def _full_spec(shape):
  return pl.BlockSpec(shape, lambda i: (0,) * len(shape))


def _row_spec(cols):
  return pl.BlockSpec((BLK, cols), lambda i: (i, 0))


def _blockdiag(att):
  H, Cc = att.shape
  eye = jnp.eye(16, dtype=att.dtype)[:H]
  return (att[:, :, None] * eye[:, None, :]).reshape(H * Cc, 16)


def kernel(x, edge_index, W1, att_src1, att_dst1, b1,
           W2, att_src2, att_dst2, b2):
  f32 = jnp.float32
  ar = jnp.arange(N_NODES, dtype=jnp.int32)
  padv = jnp.full((E_PAD - E_TOT,), N_NODES, dtype=jnp.int32)
  src = jnp.concatenate([edge_index[0].astype(jnp.int32), ar, padv])
  dst = jnp.concatenate([edge_index[1].astype(jnp.int32), ar, padv])

  A1s = _blockdiag(att_src1)
  A1d = _blockdiag(att_dst1)
  A2s = _blockdiag(att_src2)
  A2d = _blockdiag(att_dst2)
  E1 = jnp.concatenate([jnp.kron(jnp.eye(8, dtype=f32), jnp.ones((1, 16), f32)),
                        jnp.zeros((8, 128), f32)], axis=0)
  E2 = jnp.concatenate([jnp.ones((1, 64), f32),
                        jnp.zeros((15, 64), f32)], axis=0)
  z128 = jnp.zeros((NPAD, 128), f32)
  z64 = jnp.zeros((NPAD, 64), f32)
  z16 = jnp.zeros((NPAD, 16), f32)

  h1, asrc1, adst1, m1 = pl.pallas_call(
      _tc_a_body,
      grid=(GRID,),
      in_specs=[_row_spec(128), _full_spec((128, 128)),
                _full_spec((128, 16)), _full_spec((128, 16))],
      out_specs=[_row_spec(128), _row_spec(16), _row_spec(16),
                 _full_spec((2, 16))],
      out_shape=[jax.ShapeDtypeStruct((N_NODES, 128), f32),
                 jax.ShapeDtypeStruct((N_NODES, 16), f32),
                 jax.ShapeDtypeStruct((N_NODES, 16), f32),
                 jax.ShapeDtypeStruct((2, 16), f32)],
  )(x, W1, A1s, A1d)

  pad16 = ((0, 16), (0, 0))
  acc1, den1 = _edge128(jnp.pad(h1, pad16), jnp.pad(asrc1, pad16),
                        jnp.pad(adst1, pad16), m1, src, dst, z128, z16)

  h2, asrc2, adst2, m2 = pl.pallas_call(
      _tc_b_body,
      grid=(GRID,),
      in_specs=[_row_spec(128), _row_spec(128), _row_spec(16), _row_spec(16),
                _full_spec((16, 128)), _full_spec((1, 128)),
                _full_spec((128, 64)), _full_spec((64, 16)),
                _full_spec((64, 16))],
      out_specs=[_row_spec(64), _row_spec(16), _row_spec(16),
                 _full_spec((2, 16))],
      out_shape=[jax.ShapeDtypeStruct((N_NODES, 64), f32),
                 jax.ShapeDtypeStruct((N_NODES, 16), f32),
                 jax.ShapeDtypeStruct((N_NODES, 16), f32),
                 jax.ShapeDtypeStruct((2, 16), f32)],
  )(acc1[0, :N_NODES], acc1[1, :N_NODES],
    den1[0, :N_NODES], den1[1, :N_NODES],
    E1, b1.reshape(1, 128), W2, A2s, A2d)

  acc2, den2 = _edge64(jnp.pad(h2, pad16), jnp.pad(asrc2, pad16),
                       jnp.pad(adst2, pad16), m2, src, dst, z64, z16)

  out = pl.pallas_call(
      _tc_c_body,
      grid=(GRID,),
      in_specs=[_row_spec(64), _row_spec(64), _row_spec(16), _row_spec(16),
                _full_spec((16, 64)), _full_spec((1, 64))],
      out_specs=_row_spec(64),
      out_shape=jax.ShapeDtypeStruct((N_NODES, 64), f32),
  )(acc2[0, :N_NODES], acc2[1, :N_NODES],
    den2[0, :N_NODES], den2[1, :N_NODES],
    E2, b2.reshape(1, 64))
  return out
